# baseline (speedup 1.0000x reference)
import functools

import jax
import jax.numpy as jnp
from jax import lax
from jax.experimental import pallas as pl
from jax.experimental.pallas import tpu as pltpu
from jax.experimental.pallas import tpu_sc as plsc

BATCH = 16384
HIDDEN = 64
RNUM = 5

try:
    _info = plsc.get_sparse_core_info()
    _NC, _NS = _info.num_cores, _info.num_subcores
except Exception:
    _NC, _NS = 2, 16
_NW = _NC * _NS
_BPW = BATCH // _NW

_mesh = plsc.VectorSubcoreMesh(core_axis_name="c", subcore_axis_name="s")


@functools.partial(
    pl.kernel,
    mesh=_mesh,
    out_type=[
        jax.ShapeDtypeStruct((BATCH, HIDDEN), jnp.float32),
        jax.ShapeDtypeStruct((BATCH, HIDDEN), jnp.float32),
    ],
    scratch_types=[
        pltpu.VMEM((_BPW,), jnp.int32),
        pltpu.VMEM((_BPW,), jnp.int32),
        pltpu.VMEM((_BPW // 2, HIDDEN), jnp.float32),
        pltpu.VMEM((_BPW // 2, HIDDEN), jnp.float32),
        pltpu.SemaphoreType.DMA,
    ],
)
def _sc_gather(table_hbm, uidx_hbm, midx_hbm, outu_hbm, outm_hbm,
               uidx_vm, midx_vm, rowsu_v, rowsm_v, sem):
    wid = lax.axis_index("s") * _NC + lax.axis_index("c")
    obase = wid * _BPW
    half = _BPW // 2
    pltpu.sync_copy(uidx_hbm.at[pl.ds(obase, _BPW)], uidx_vm)
    pltpu.sync_copy(midx_hbm.at[pl.ds(obase, _BPW)], midx_vm)

    for ph in range(2):
        pbase = ph * half

        def body(g, carry):
            vu = uidx_vm[pl.ds(pbase + g * 16, 16)]
            vm_ = midx_vm[pl.ds(pbase + g * 16, 16)]
            for k in range(16):
                pltpu.async_copy(table_hbm.at[pl.ds(vu[k], 1)],
                                 rowsu_v.at[pl.ds(g * 16 + k, 1)], sem)
                pltpu.async_copy(table_hbm.at[pl.ds(vm_[k], 1)],
                                 rowsm_v.at[pl.ds(g * 16 + k, 1)], sem)
            return carry

        lax.fori_loop(0, half // 16, body, 0)
        pltpu.make_async_copy(table_hbm.at[pl.ds(0, half)], rowsu_v, sem).wait()
        pltpu.make_async_copy(table_hbm.at[pl.ds(0, half)], rowsm_v, sem).wait()
        pltpu.sync_copy(rowsu_v, outu_hbm.at[pl.ds(obase + pbase, half)])
        pltpu.sync_copy(rowsm_v, outm_hbm.at[pl.ds(obase + pbase, half)])


_BM = 2048


def _mlp_body(xu_ref, xm_ref, w1_ref, b1_ref, w2_ref, b2_ref, out_ref):
    dn = (((1,), (0,)), ((), ()))
    hi = jax.lax.Precision.HIGHEST
    pre = (
        lax.dot_general(xu_ref[...], w1_ref[0:HIDDEN, :], dn,
                        precision=hi, preferred_element_type=jnp.float32)
        + lax.dot_general(xm_ref[...], w1_ref[HIDDEN:2 * HIDDEN, :], dn,
                          precision=hi, preferred_element_type=jnp.float32)
        + b1_ref[...]
    )
    h = jnp.tanh(pre)
    out_ref[...] = (
        lax.dot_general(h, w2_ref[...], dn,
                        precision=hi, preferred_element_type=jnp.float32)
        + b2_ref[...]
    )


_tc_mlp = pl.pallas_call(
    _mlp_body,
    grid=(BATCH // _BM,),
    in_specs=[
        pl.BlockSpec((_BM, HIDDEN), lambda i: (i, 0)),
        pl.BlockSpec((_BM, HIDDEN), lambda i: (i, 0)),
        pl.BlockSpec((2 * HIDDEN, HIDDEN), lambda i: (0, 0)),
        pl.BlockSpec((1, HIDDEN), lambda i: (0, 0)),
        pl.BlockSpec((HIDDEN, RNUM), lambda i: (0, 0)),
        pl.BlockSpec((1, RNUM), lambda i: (0, 0)),
    ],
    out_specs=pl.BlockSpec((_BM, RNUM), lambda i: (i, 0)),
    out_shape=jax.ShapeDtypeStruct((BATCH, RNUM), jnp.float32),
)


def kernel(data, movie_table, user_table, W1, b1, W2, b2):
    uidx = data[:, 0].astype(jnp.int32)
    midx = data[:, 1].astype(jnp.int32)
    xu, xm = _sc_gather(movie_table, uidx, midx)
    return _tc_mlp(xu, xm, W1, b1.reshape(1, HIDDEN), W2, b2.reshape(1, RNUM))


# speedup vs baseline: 1.0582x; 1.0582x over previous
import functools

import jax
import jax.numpy as jnp
from jax import lax
from jax.experimental import pallas as pl
from jax.experimental.pallas import tpu as pltpu
from jax.experimental.pallas import tpu_sc as plsc

BATCH = 16384
HIDDEN = 64
RNUM = 5

try:
    _info = plsc.get_sparse_core_info()
    _NC, _NS = _info.num_cores, _info.num_subcores
except Exception:
    _NC, _NS = 2, 16
_NW = _NC * _NS
_BPW = BATCH // _NW

_mesh = plsc.VectorSubcoreMesh(core_axis_name="c", subcore_axis_name="s")


@functools.partial(
    pl.kernel,
    mesh=_mesh,
    out_type=[
        jax.ShapeDtypeStruct((BATCH, HIDDEN), jnp.float32),
        jax.ShapeDtypeStruct((BATCH, HIDDEN), jnp.float32),
    ],
    scratch_types=[
        pltpu.VMEM((_BPW,), jnp.int32),
        pltpu.VMEM((_BPW,), jnp.int32),
        pltpu.VMEM((_BPW // 2, HIDDEN), jnp.float32),
        pltpu.VMEM((_BPW // 2, HIDDEN), jnp.float32),
        pltpu.SemaphoreType.DMA,
    ],
)
def _sc_gather(table_hbm, uidx_hbm, midx_hbm, outu_hbm, outm_hbm,
               uidx_vm, midx_vm, rowsu_v, rowsm_v, sem):
    wid = lax.axis_index("s") * _NC + lax.axis_index("c")
    obase = wid * _BPW
    half = _BPW // 2
    pltpu.sync_copy(uidx_hbm.at[pl.ds(obase, _BPW)], uidx_vm)
    pltpu.sync_copy(midx_hbm.at[pl.ds(obase, _BPW)], midx_vm)

    for ph in range(2):
        pbase = ph * half

        def body(g, carry):
            vu = uidx_vm[pl.ds(pbase + g * 16, 16)]
            vm_ = midx_vm[pl.ds(pbase + g * 16, 16)]
            for k in range(16):
                pltpu.async_copy(table_hbm.at[pl.ds(vu[k], 1)],
                                 rowsu_v.at[pl.ds(g * 16 + k, 1)], sem)
                pltpu.async_copy(table_hbm.at[pl.ds(vm_[k], 1)],
                                 rowsm_v.at[pl.ds(g * 16 + k, 1)], sem)
            return carry

        lax.fori_loop(0, half // 16, body, 0)
        pltpu.make_async_copy(table_hbm.at[pl.ds(0, half)], rowsu_v, sem).wait()
        pltpu.make_async_copy(table_hbm.at[pl.ds(0, half)], rowsm_v, sem).wait()
        pltpu.sync_copy(rowsu_v, outu_hbm.at[pl.ds(obase + pbase, half)])
        pltpu.sync_copy(rowsm_v, outm_hbm.at[pl.ds(obase + pbase, half)])


_BM = 2048


def _mlp_body(xu_ref, xm_ref, w1_ref, b1_ref, w2_ref, b2_ref, out_ref):
    dn = (((1,), (0,)), ((), ()))
    hi = None
    pre = (
        lax.dot_general(xu_ref[...], w1_ref[0:HIDDEN, :], dn,
                        precision=hi, preferred_element_type=jnp.float32)
        + lax.dot_general(xm_ref[...], w1_ref[HIDDEN:2 * HIDDEN, :], dn,
                          precision=hi, preferred_element_type=jnp.float32)
        + b1_ref[...]
    )
    h = jnp.tanh(pre)
    out_ref[...] = (
        lax.dot_general(h, w2_ref[...], dn,
                        precision=hi, preferred_element_type=jnp.float32)
        + b2_ref[...]
    )


_tc_mlp = pl.pallas_call(
    _mlp_body,
    grid=(BATCH // _BM,),
    in_specs=[
        pl.BlockSpec((_BM, HIDDEN), lambda i: (i, 0)),
        pl.BlockSpec((_BM, HIDDEN), lambda i: (i, 0)),
        pl.BlockSpec((2 * HIDDEN, HIDDEN), lambda i: (0, 0)),
        pl.BlockSpec((1, HIDDEN), lambda i: (0, 0)),
        pl.BlockSpec((HIDDEN, RNUM), lambda i: (0, 0)),
        pl.BlockSpec((1, RNUM), lambda i: (0, 0)),
    ],
    out_specs=pl.BlockSpec((_BM, RNUM), lambda i: (i, 0)),
    out_shape=jax.ShapeDtypeStruct((BATCH, RNUM), jnp.float32),
)


def kernel(data, movie_table, user_table, W1, b1, W2, b2):
    uidx = data[:, 0].astype(jnp.int32)
    midx = data[:, 1].astype(jnp.int32)
    xu, xm = _sc_gather(movie_table, uidx, midx)
    return _tc_mlp(xu, xm, W1, b1.reshape(1, HIDDEN), W2, b2.reshape(1, RNUM))
